# baseline (speedup 1.0000x reference)
import jax
import jax.numpy as jnp
from jax.experimental import pallas as pl
from jax.experimental.pallas import tpu as pltpu


_NCHUNKS = 16


def _copy_body(x_hbm, o_hbm, sems):
    rows = x_hbm.shape[0] // _NCHUNKS
    for i in range(_NCHUNKS):
        sl = pl.ds(i * rows, rows)
        pltpu.make_async_copy(x_hbm.at[sl], o_hbm.at[sl], sems.at[i]).start()
    for i in range(_NCHUNKS):
        sl = pl.ds(i * rows, rows)
        pltpu.make_async_copy(x_hbm.at[sl], o_hbm.at[sl], sems.at[i]).wait()


def kernel(x):
    return pl.pallas_call(
        _copy_body,
        out_shape=jax.ShapeDtypeStruct(x.shape, x.dtype),
        in_specs=[pl.BlockSpec(memory_space=pl.ANY)],
        out_specs=pl.BlockSpec(memory_space=pl.ANY),
        scratch_shapes=[pltpu.SemaphoreType.DMA((_NCHUNKS,))],
    )(x)


# speedup vs baseline: 27.8663x; 27.8663x over previous
import jax
import jax.numpy as jnp
from jax.experimental import pallas as pl
from jax.experimental.pallas import tpu as pltpu


_BLOCK_ROWS = 2048


def _copy_body(x_ref, o_ref):
    o_ref[...] = x_ref[...]


def kernel(x):
    rows, cols = x.shape
    grid = (rows // _BLOCK_ROWS,)
    spec = pl.BlockSpec((_BLOCK_ROWS, cols), lambda i: (i, 0))
    return pl.pallas_call(
        _copy_body,
        out_shape=jax.ShapeDtypeStruct(x.shape, x.dtype),
        grid=grid,
        in_specs=[spec],
        out_specs=spec,
    )(x)


# speedup vs baseline: 34.7330x; 1.2464x over previous
import jax
import jax.numpy as jnp
from jax.experimental import pallas as pl
from jax.experimental.pallas import tpu as pltpu


_BLOCK_ROWS = 4096


def _copy_body(x_ref, o_ref):
    o_ref[...] = x_ref[...]


def kernel(x):
    rows, cols = x.shape
    grid = (rows // _BLOCK_ROWS,)
    spec = pl.BlockSpec((_BLOCK_ROWS, cols), lambda i: (i, 0))
    return pl.pallas_call(
        _copy_body,
        out_shape=jax.ShapeDtypeStruct(x.shape, x.dtype),
        grid=grid,
        in_specs=[spec],
        out_specs=spec,
    )(x)


# speedup vs baseline: 42.3141x; 1.2183x over previous
import jax
import jax.numpy as jnp
from jax.experimental import pallas as pl
from jax.experimental.pallas import tpu as pltpu


_BLOCK_ROWS = 8192


def _copy_body(x_ref, o_ref):
    o_ref[...] = x_ref[...]


def kernel(x):
    rows, cols = x.shape
    grid = (rows // _BLOCK_ROWS,)
    spec = pl.BlockSpec((_BLOCK_ROWS, cols), lambda i: (i, 0))
    return pl.pallas_call(
        _copy_body,
        out_shape=jax.ShapeDtypeStruct(x.shape, x.dtype),
        grid=grid,
        in_specs=[spec],
        out_specs=spec,
    )(x)
